# baseline (speedup 1.0000x reference)
import jax
import jax.numpy as jnp
from jax.experimental import pallas as pl
from jax.experimental.pallas import tpu as pltpu

_BF = jnp.bfloat16
_F32 = jnp.float32
_PARALLEL = pltpu.CompilerParams(dimension_semantics=("parallel",))


def _conv_pool_kernel(p_ref, w_ref, b_ref, o_ref):
    w = w_ref[...]

    def mm(off):
        return jnp.dot(w, p_ref[off], preferred_element_type=_F32)

    m = jnp.maximum(jnp.maximum(mm(0), mm(1)), jnp.maximum(mm(2), mm(3)))
    o_ref[...] = jnp.maximum(m + b_ref[...], 0.0).astype(_BF)


def _head_kernel(f_ref, w1_ref, b1_ref, w2_ref, b2_ref, w3_ref, b3_ref, o_ref):
    h1 = jnp.dot(w1_ref[...], f_ref[...], preferred_element_type=_F32)
    h1 = jnp.maximum(h1 + b1_ref[...], 0.0).astype(_BF)
    h2 = jnp.dot(w2_ref[...], h1, preferred_element_type=_F32)
    h2 = jnp.maximum(h2 + b2_ref[...], 0.0).astype(_BF)
    lg = jnp.dot(w3_ref[...], h2, preferred_element_type=_F32) + b3_ref[...]
    mx = jnp.max(lg, axis=0, keepdims=True)
    e = jnp.exp(lg - mx)
    o_ref[...] = e * pl.reciprocal(jnp.sum(e, axis=0, keepdims=True), approx=True)


def _pooled_patches(x, k, n_pool, col_major_pos):
    bsz, c = x.shape[0], x.shape[1]
    span = 2 * n_pool - 1
    offs = []
    for dh in range(2):
        for dw in range(2):
            cols = []
            for kh in range(k):
                for kw in range(k):
                    sl = x[:, :, dh + kh:dh + kh + span:2, dw + kw:dw + kw + span:2]
                    cols.append(sl)
            p = jnp.stack(cols, axis=2)
            if col_major_pos:
                p = p.transpose(1, 2, 3, 4, 0)
            else:
                p = p.transpose(1, 2, 0, 3, 4)
            offs.append(p.reshape(c * k * k, bsz * n_pool * n_pool))
    return jnp.stack(offs, axis=0).astype(_BF)


def _conv_pool(pats, w, b, co, grid):
    ncols = pats.shape[2]
    nc = ncols // grid
    taps = pats.shape[1]
    return pl.pallas_call(
        _conv_pool_kernel,
        out_shape=jax.ShapeDtypeStruct((co, ncols), _BF),
        grid=(grid,),
        in_specs=[pl.BlockSpec((4, taps, nc), lambda g: (0, 0, g)),
                  pl.BlockSpec((co, taps), lambda g: (0, 0)),
                  pl.BlockSpec((co, 1), lambda g: (0, 0))],
        out_specs=pl.BlockSpec((co, nc), lambda g: (0, g)),
        compiler_params=_PARALLEL,
    )(pats, w, b)


def kernel(conv1, conv2, fc1_w, head, x):
    bsz = x.shape[0]

    w1c = conv1[:, 0:75].astype(_BF)
    b1c = conv1[:, 75:76]
    w2c = conv2[:, 0:150].astype(_BF)
    b2c = conv2[:, 150:151]
    w1 = fc1_w.reshape(400, 120).T.astype(_BF)
    bf1 = head[0:1, 0:120].T
    w2 = head[8:128, 0:84].T.astype(_BF)
    bf2 = head[128:129, 0:84].T
    w3 = head[136:220, 0:4].T.astype(_BF)
    bf3 = head[224:225, 0:4].T

    pats1 = _pooled_patches(x, 5, 14, col_major_pos=False)
    pool1 = _conv_pool(pats1, w1c, b1c, 6, grid=32)

    y = pool1.reshape(6, bsz, 14, 14).transpose(1, 0, 2, 3)
    pats2 = _pooled_patches(y, 5, 5, col_major_pos=True)
    pool2 = _conv_pool(pats2, w2c, b2c, 16, grid=8)

    feats = pool2.reshape(400, bsz)
    nb = bsz // 4
    probs = pl.pallas_call(
        _head_kernel,
        out_shape=jax.ShapeDtypeStruct((4, bsz), _F32),
        grid=(4,),
        in_specs=[pl.BlockSpec((400, nb), lambda g: (0, g)),
                  pl.BlockSpec((120, 400), lambda g: (0, 0)),
                  pl.BlockSpec((120, 1), lambda g: (0, 0)),
                  pl.BlockSpec((84, 120), lambda g: (0, 0)),
                  pl.BlockSpec((84, 1), lambda g: (0, 0)),
                  pl.BlockSpec((4, 84), lambda g: (0, 0)),
                  pl.BlockSpec((4, 1), lambda g: (0, 0))],
        out_specs=pl.BlockSpec((4, nb), lambda g: (0, g)),
        compiler_params=_PARALLEL,
    )(feats, w1, bf1, w2, bf2, w3, bf3)
    return probs.T


# speedup vs baseline: 31.9535x; 31.9535x over previous
import jax
import jax.numpy as jnp
from jax.experimental import pallas as pl
from jax.experimental.pallas import tpu as pltpu

_BF = jnp.bfloat16
_F32 = jnp.float32
_PARALLEL = pltpu.CompilerParams(dimension_semantics=("parallel",))


def _conv_pool_kernel(p_ref, w_ref, b_ref, o_ref):
    w = w_ref[...]

    def mm(off):
        return jnp.dot(w, p_ref[off], preferred_element_type=_F32)

    m = jnp.maximum(jnp.maximum(mm(0), mm(1)), jnp.maximum(mm(2), mm(3)))
    o_ref[...] = jnp.maximum(m + b_ref[...], 0.0).astype(_BF)


def _head_kernel(f_ref, w1_ref, b1_ref, w2_ref, b2_ref, w3_ref, b3_ref, o_ref):
    h1 = jnp.dot(w1_ref[...], f_ref[...], preferred_element_type=_F32)
    h1 = jnp.maximum(h1 + b1_ref[...], 0.0).astype(_BF)
    h2 = jnp.dot(w2_ref[...], h1, preferred_element_type=_F32)
    h2 = jnp.maximum(h2 + b2_ref[...], 0.0).astype(_BF)
    lg = jnp.dot(w3_ref[...], h2, preferred_element_type=_F32) + b3_ref[...]
    mx = jnp.max(lg, axis=0, keepdims=True)
    e = jnp.exp(lg - mx)
    o_ref[...] = e * pl.reciprocal(jnp.sum(e, axis=0, keepdims=True), approx=True)


def _pooled_patches(xt, k, n_pool):
    c, bsz = xt.shape[0], xt.shape[3]
    span = 2 * n_pool - 1
    offs = []
    for dh in range(2):
        for dw in range(2):
            cols = []
            for kh in range(k):
                for kw in range(k):
                    sl = xt[:, dh + kh:dh + kh + span:2, dw + kw:dw + kw + span:2, :]
                    cols.append(sl)
            p = jnp.stack(cols, axis=1)
            offs.append(p.reshape(c * k * k, n_pool * n_pool * bsz))
    return jnp.stack(offs, axis=0).astype(_BF)


def _conv_pool(pats, w, b, co, grid):
    ncols = pats.shape[2]
    nc = ncols // grid
    taps = pats.shape[1]
    return pl.pallas_call(
        _conv_pool_kernel,
        out_shape=jax.ShapeDtypeStruct((co, ncols), _BF),
        grid=(grid,),
        in_specs=[pl.BlockSpec((4, taps, nc), lambda g: (0, 0, g)),
                  pl.BlockSpec((co, taps), lambda g: (0, 0)),
                  pl.BlockSpec((co, 1), lambda g: (0, 0))],
        out_specs=pl.BlockSpec((co, nc), lambda g: (0, g)),
        compiler_params=_PARALLEL,
    )(pats, w, b)


def kernel(conv1, conv2, fc1_w, head, x):
    bsz = x.shape[0]

    w1c = conv1[:, 0:75].astype(_BF)
    b1c = conv1[:, 75:76]
    w2c = conv2[:, 0:150].astype(_BF)
    b2c = conv2[:, 150:151]
    w1 = fc1_w.reshape(400, 120).T.astype(_BF)
    bf1 = head[0:1, 0:120].T
    w2 = head[8:128, 0:84].T.astype(_BF)
    bf2 = head[128:129, 0:84].T
    w3 = head[136:220, 0:4].T.astype(_BF)
    bf3 = head[224:225, 0:4].T

    xt = x.transpose(1, 2, 3, 0)
    pats1 = _pooled_patches(xt, 5, 14)
    pool1 = _conv_pool(pats1, w1c, b1c, 6, grid=32)

    y = pool1.reshape(6, 14, 14, bsz)
    pats2 = _pooled_patches(y, 5, 5)
    pool2 = _conv_pool(pats2, w2c, b2c, 16, grid=8)

    feats = pool2.reshape(400, bsz)
    nb = bsz // 4
    probs = pl.pallas_call(
        _head_kernel,
        out_shape=jax.ShapeDtypeStruct((4, bsz), _F32),
        grid=(4,),
        in_specs=[pl.BlockSpec((400, nb), lambda g: (0, g)),
                  pl.BlockSpec((120, 400), lambda g: (0, 0)),
                  pl.BlockSpec((120, 1), lambda g: (0, 0)),
                  pl.BlockSpec((84, 120), lambda g: (0, 0)),
                  pl.BlockSpec((84, 1), lambda g: (0, 0)),
                  pl.BlockSpec((4, 84), lambda g: (0, 0)),
                  pl.BlockSpec((4, 1), lambda g: (0, 0))],
        out_specs=pl.BlockSpec((4, nb), lambda g: (0, g)),
        compiler_params=_PARALLEL,
    )(feats, w1, bf1, w2, bf2, w3, bf3)
    return probs.T


# speedup vs baseline: 182.3233x; 5.7059x over previous
import jax
import jax.numpy as jnp
from jax.experimental import pallas as pl
from jax.experimental.pallas import tpu as pltpu

_BF = jnp.bfloat16
_F32 = jnp.float32


def _pool2x2(a, n):
    a = jnp.max(a.reshape(n, 2, 2 * n, a.shape[-1]), axis=1)
    a = jnp.max(a.reshape(n, n, 2, a.shape[-1]), axis=2)
    return a


def _fused_kernel(xt_ref, c1_ref, c2_ref, w1_ref, b1_ref, w2_ref, b2_ref,
                  w3_ref, b3_ref, o_ref, sx_ref, p1_ref, sx2_ref, f_ref):
    nb = xt_ref.shape[-1]
    xv = xt_ref[...]
    for kw in range(5):
        sx_ref[kw] = xv[:, :, kw:kw + 28, :]

    def c1_body(co, _):
        acc = jnp.full((28, 28, nb), c1_ref[co, 75], _F32)
        for ci in range(3):
            for kh in range(5):
                for kw in range(5):
                    acc = acc + c1_ref[co, ci * 25 + kh * 5 + kw] * sx_ref[kw, ci, kh:kh + 28]
        p1_ref[co] = _pool2x2(jnp.maximum(acc, 0.0), 14)
        return 0

    jax.lax.fori_loop(0, 6, c1_body, 0, unroll=False)

    for kw in range(5):
        for ci in range(6):
            sx2_ref[kw, ci] = p1_ref[ci][:, kw:kw + 10, :]

    f_ref[:, :, 5:8, :] = jnp.zeros((16, 5, 3, nb), _BF)

    def c2_body(co, _):
        acc = jnp.full((10, 10, nb), c2_ref[co, 150], _F32)
        for ci in range(6):
            for kh in range(5):
                for kw in range(5):
                    acc = acc + c2_ref[co, ci * 25 + kh * 5 + kw] * sx2_ref[kw, ci, kh:kh + 10]
        f_ref[co, :, 0:5, :] = _pool2x2(jnp.maximum(acc, 0.0), 5).astype(_BF)
        return 0

    jax.lax.fori_loop(0, 16, c2_body, 0, unroll=False)

    feats = f_ref[...].reshape(640, xv.shape[-1])
    h1 = jnp.dot(w1_ref[...], feats, preferred_element_type=_F32)
    h1 = jnp.maximum(h1 + b1_ref[...], 0.0).astype(_BF)
    h2 = jnp.dot(w2_ref[...], h1, preferred_element_type=_F32)
    h2 = jnp.maximum(h2 + b2_ref[...], 0.0).astype(_BF)
    lg = jnp.dot(w3_ref[...], h2, preferred_element_type=_F32) + b3_ref[...]
    mx = jnp.max(lg, axis=0, keepdims=True)
    e = jnp.exp(lg - mx)
    o_ref[...] = e * pl.reciprocal(jnp.sum(e, axis=0, keepdims=True), approx=True)


def kernel(conv1, conv2, fc1_w, head, x):
    bsz = x.shape[0]
    nb = 128
    grid = bsz // nb

    w1 = fc1_w.transpose(2, 0, 1).reshape(120, 16, 5, 5)
    w1 = jnp.pad(w1, ((0, 0), (0, 0), (0, 0), (0, 3))).reshape(120, 640).astype(_BF)
    bf1 = head[0:1, 0:120].T
    w2 = head[8:128, 0:84].T.astype(_BF)
    bf2 = head[128:129, 0:84].T
    w3 = head[136:220, 0:4].T.astype(_BF)
    bf3 = head[224:225, 0:4].T

    xt = x.transpose(1, 2, 3, 0)

    probs = pl.pallas_call(
        _fused_kernel,
        out_shape=jax.ShapeDtypeStruct((4, bsz), _F32),
        grid=(grid,),
        in_specs=[pl.BlockSpec((3, 32, 32, nb), lambda g: (0, 0, 0, g)),
                  pl.BlockSpec(memory_space=pltpu.SMEM),
                  pl.BlockSpec(memory_space=pltpu.SMEM),
                  pl.BlockSpec((120, 640), lambda g: (0, 0)),
                  pl.BlockSpec((120, 1), lambda g: (0, 0)),
                  pl.BlockSpec((84, 120), lambda g: (0, 0)),
                  pl.BlockSpec((84, 1), lambda g: (0, 0)),
                  pl.BlockSpec((4, 84), lambda g: (0, 0)),
                  pl.BlockSpec((4, 1), lambda g: (0, 0))],
        out_specs=pl.BlockSpec((4, nb), lambda g: (0, g)),
        scratch_shapes=[pltpu.VMEM((5, 3, 32, 28, nb), _F32),
                        pltpu.VMEM((6, 14, 14, nb), _F32),
                        pltpu.VMEM((5, 6, 14, 10, nb), _F32),
                        pltpu.VMEM((16, 5, 8, nb), _BF)],
        compiler_params=pltpu.CompilerParams(
            dimension_semantics=("parallel",)),
    )(xt, conv1, conv2, w1, bf1, w2, bf2, w3, bf3)
    return probs.T
